# baseline (speedup 1.0000x reference)
import functools

import jax
import jax.numpy as jnp
from jax import lax
from jax.experimental import pallas as pl
from jax.experimental.pallas import tpu as pltpu
from jax.experimental.pallas import tpu_sc as plsc

N_STATES = 1000000
EMBED_DIM = 64
WIDE = 128
HIDDEN_DIM = 128
N_ACTIONS = 18
BATCH = 16384

NC = 2
NS = 16
NW = NC * NS
B_PER_W = BATCH // NW
CHUNK = 128
NCHUNK = B_PER_W // CHUNK

WS = 32768
WSH = 15
Q = 8192
QH = 13
NBLK = (N_STATES + WS - 1) // WS
W4ROWS = NBLK * Q

HI16 = -65536


def _widen_body(in_ref, out_ref):
    x = lax.bitcast_convert_type(in_ref[...], jnp.int32)
    p0 = jnp.bitwise_or(jnp.bitwise_and(x[:, :Q], HI16),
                        lax.shift_right_logical(x[:, Q:2 * Q], 16))
    p1 = jnp.bitwise_or(jnp.bitwise_and(x[:, 2 * Q:3 * Q], HI16),
                        lax.shift_right_logical(x[:, 3 * Q:], 16))
    out_ref[:, :EMBED_DIM] = lax.bitcast_convert_type(p0.T, jnp.float32)
    out_ref[:, EMBED_DIM:] = lax.bitcast_convert_type(p1.T, jnp.float32)


def _tc_widen(table_t):
    grid = (NBLK,)
    return pl.pallas_call(
        _widen_body,
        grid=grid,
        in_specs=[pl.BlockSpec((EMBED_DIM, WS), lambda i: (0, i))],
        out_specs=pl.BlockSpec((Q, WIDE), lambda i: (i, 0)),
        out_shape=jax.ShapeDtypeStruct((W4ROWS, WIDE), jnp.float32),
        compiler_params=pltpu.CompilerParams(
            dimension_semantics=("arbitrary",),
        ),
    )(table_t)


def _sc_gather_wide(s2, table2):
    mesh = plsc.VectorSubcoreMesh(core_axis_name="c", subcore_axis_name="s",
                                  num_cores=NC, num_subcores=NS)

    @functools.partial(
        pl.kernel,
        out_type=jax.ShapeDtypeStruct((BATCH, WIDE), jnp.float32),
        mesh=mesh,
        scratch_types=[
            pltpu.VMEM((B_PER_W,), jnp.int32),
            pltpu.VMEM((B_PER_W, WIDE), jnp.float32),
            pltpu.SemaphoreType.DMA,
        ],
    )
    def gather_kernel(s_hbm, table_hbm, out_hbm, idx_v, rows_v, sem):
        wid = lax.axis_index("s") * NC + lax.axis_index("c")
        base = wid * B_PER_W
        pltpu.sync_copy(s_hbm.at[pl.ds(base, B_PER_W)], idx_v)
        copies = []
        for j in range(NCHUNK):
            copies.append(pltpu.make_async_copy(
                table_hbm.at[idx_v.at[pl.ds(j * CHUNK, CHUNK)]],
                rows_v.at[pl.ds(j * CHUNK, CHUNK)],
                sem))
        for c in copies:
            c.start()
        for c in copies:
            c.wait()
        pltpu.sync_copy(rows_v, out_hbm.at[pl.ds(base, B_PER_W)])

    return gather_kernel(s2, table2)


def _mlp_body(xw_ref, sel_ref, w1_ref, b1_ref, w2_ref, b2_ref,
              w3_ref, b3_ref, o_ref):
    xi = lax.bitcast_convert_type(xw_ref[...], jnp.int32)
    va = lax.bitcast_convert_type(jnp.bitwise_and(xi, HI16), jnp.float32)
    vb = lax.bitcast_convert_type(lax.shift_left(xi, 16), jnp.float32)
    t_row = jnp.squeeze(sel_ref[...], axis=0)
    t_col = jnp.transpose(t_row)
    v = jnp.where(jnp.bitwise_and(t_col, 1) == 0, va, vb)
    lane = lax.broadcasted_iota(jnp.int32, v.shape, 1)
    xm = jnp.where((lane < EMBED_DIM) == (t_col < 2), v, 0.0)
    h = jnp.dot(xm, w1_ref[...], preferred_element_type=jnp.float32)
    h = jnp.maximum(h + b1_ref[...], 0.0)
    h = jnp.dot(h, w2_ref[...], preferred_element_type=jnp.float32)
    h = jnp.maximum(h + b2_ref[...], 0.0)
    o = jnp.dot(h, w3_ref[...], preferred_element_type=jnp.float32)
    o_ref[...] = o + b3_ref[...]


def _tc_mlp(xw, sel3, W1s, b1, W2, b2, W3, b3, blk=4096, interpret=False):
    grid = (BATCH // blk,)
    return pl.pallas_call(
        _mlp_body,
        grid=grid,
        in_specs=[
            pl.BlockSpec((blk, WIDE), lambda i: (i, 0)),
            pl.BlockSpec((1, 1, blk), lambda i: (i, 0, 0)),
            pl.BlockSpec((2 * EMBED_DIM, HIDDEN_DIM), lambda i: (0, 0)),
            pl.BlockSpec((1, HIDDEN_DIM), lambda i: (0, 0)),
            pl.BlockSpec((HIDDEN_DIM, HIDDEN_DIM), lambda i: (0, 0)),
            pl.BlockSpec((1, HIDDEN_DIM), lambda i: (0, 0)),
            pl.BlockSpec((HIDDEN_DIM, N_ACTIONS), lambda i: (0, 0)),
            pl.BlockSpec((1, N_ACTIONS), lambda i: (0, 0)),
        ],
        out_specs=pl.BlockSpec((blk, N_ACTIONS), lambda i: (i, 0)),
        out_shape=jax.ShapeDtypeStruct((BATCH, N_ACTIONS), jnp.float32),
        compiler_params=pltpu.CompilerParams(
            dimension_semantics=("arbitrary",),
        ),
        interpret=interpret,
    )(xw, sel3, W1s, b1.reshape(1, -1), W2, b2.reshape(1, -1),
      W3, b3.reshape(1, -1))


def kernel(s, table, W1, b1, W2, b2, W3, b3):
    s32 = s.astype(jnp.int32)
    s2 = jnp.bitwise_or(
        lax.shift_left(lax.shift_right_logical(s32, WSH), QH),
        jnp.bitwise_and(s32, Q - 1))
    t = jnp.bitwise_and(lax.shift_right_logical(s32, QH), 3)
    sel3 = t.reshape(BATCH // 4096, 1, 4096)
    table2 = _tc_widen(table.T)
    xw = _sc_gather_wide(s2, table2)
    W1s = jnp.concatenate([W1, W1], axis=0)
    return _tc_mlp(xw, sel3, W1s, b1, W2, b2, W3, b3)
